# Initial kernel scaffold; baseline (speedup 1.0000x reference)
import jax
import jax.numpy as jnp
from jax import lax
from jax.experimental import pallas as pl
from jax.experimental.pallas import tpu as pltpu
from jax.experimental.pallas import tpu_sc as plsc

N_POOLED, N, E, C = 5000, 10000, 320000, 128
NC, NS, L = 2, 16, 16
CH = C // NC
NP = 10240
NPP = 5120
NROW = NP // 128
EROW = E // 128
PAD_IDX = N_POOLED

_mesh = plsc.VectorSubcoreMesh(
    core_axis_name="c", subcore_axis_name="s", num_cores=NC, num_subcores=NS)


def _hist_body(col_hbm, clus_hbm, fdeg_hbm, fcc_hbm,
               histd_sp, histc_sp, ibuf, clbuf, ones_v, zero_v):
    c = lax.axis_index("c")
    s = lax.axis_index("s")
    for k in range(8):
        ones_v[pl.ds(16 * k, 16)] = jnp.full((16,), 1.0, jnp.float32)

    def _zf(i, carry):
        zero_v[pl.ds(16 * i, 16)] = jnp.zeros((16,), jnp.float32)
        return carry

    lax.fori_loop(0, 40, _zf, 0)
    pltpu.sync_copy(zero_v, histd_sp.at[pl.ds(640 * s, 640)])

    @pl.when(c == 1)
    def _():
        pltpu.sync_copy(zero_v.at[pl.ds(0, 320)], histc_sp.at[pl.ds(320 * s, 320)])

    plsc.subcore_barrier()

    start = 1250 * c + 78 * s + jnp.minimum(s, 2)
    pltpu.sync_copy(col_hbm.at[pl.ds(start, 78)], ibuf.at[pl.ds(0, 78)])

    @pl.when(s < 2)
    def _():
        pltpu.sync_copy(col_hbm.at[pl.ds(start + 78, 1)], ibuf.at[pl.ds(78, 1)])

    nj = 78 + jnp.where(s < 2, 1, 0)

    def _hb(j, carry):
        pltpu.sync_copy(ones_v, histd_sp.at[ibuf.at[j]], add=True)
        return carry

    lax.fori_loop(0, nj, _hb, 0)

    @pl.when(c == 1)
    def _():
        pltpu.sync_copy(clus_hbm.at[pl.ds(5 * s, 5)], clbuf)
        for j in range(5):
            pltpu.sync_copy(ones_v, histc_sp.at[clbuf.at[j]], add=True)

    plsc.subcore_barrier()
    pltpu.sync_copy(histd_sp.at[pl.ds(640 * s, 640)],
                    fdeg_hbm.at[c, pl.ds(640 * s, 640)])

    @pl.when(c == 1)
    def _():
        pltpu.sync_copy(histc_sp.at[pl.ds(320 * s, 320)],
                        fcc_hbm.at[pl.ds(320 * s, 320)])


_hist_call = pl.kernel(
    _hist_body,
    out_type=[jax.ShapeDtypeStruct((NC, NP), jnp.float32),
              jax.ShapeDtypeStruct((NPP,), jnp.float32)],
    mesh=_mesh,
    scratch_types=[
        pltpu.VMEM_SHARED((NP,), jnp.float32),
        pltpu.VMEM_SHARED((NPP,), jnp.float32),
        pltpu.VMEM((79, 128), jnp.int32),
        pltpu.VMEM((5, 128), jnp.int32),
        pltpu.VMEM((128,), jnp.float32),
        pltpu.VMEM((640,), jnp.float32),
    ],
)


def _dense_body(x_ref, sc_ref, w1_ref, b1_ref, g1_ref, be_ref, wc_ref, cc_ref,
                f0_ref, f1_ref, zz_ref, dinvb_ref, dinv_ref):
    xsn = x_ref[...] / sc_ref[...]
    g = jnp.dot(xsn, w1_ref[...], preferred_element_type=jnp.float32) + b1_ref[...]
    cg = cc_ref[...] * g
    mean = jnp.sum(cg, axis=0, keepdims=True) * (1.0 / N)
    e2 = jnp.sum(cg * g, axis=0, keepdims=True) * (1.0 / N)
    var = e2 - mean * mean
    t = (g - mean) * lax.rsqrt(var + 1e-5) * g1_ref[...] + be_ref[...]
    t = jnp.maximum(t, 0.0)
    z = jnp.dot(t, wc_ref[...], preferred_element_type=jnp.float32)
    zz_ref[0, 0:N_POOLED, :] = z[:, 0:CH]
    zz_ref[1, 0:N_POOLED, :] = z[:, CH:C]
    zz_ref[0, N_POOLED:NPP, :] = jnp.zeros((NPP - N_POOLED, CH), jnp.float32)
    zz_ref[1, N_POOLED:NPP, :] = jnp.zeros((NPP - N_POOLED, CH), jnp.float32)
    dinv = lax.rsqrt(f0_ref[...] + f1_ref[...] + 1.0)
    dinv_ref[...] = dinv
    dinvb_ref[...] = jnp.broadcast_to(dinv, (NP, CH))


_dense_call = pl.pallas_call(
    _dense_body,
    out_shape=[jax.ShapeDtypeStruct((NC, NPP, CH), jnp.float32),
               jax.ShapeDtypeStruct((NP, CH), jnp.float32),
               jax.ShapeDtypeStruct((NP, 1), jnp.float32)],
)


def _edge_body(zz_hbm, cl_hbm, dinvb_hbm, row_hbm, col_hbm, acc_hbm,
               z_sp, y_sp, acc_sp, clbuf, dbb, rbuf, cbuf, gbuf, ybuf, egbuf,
               sem, sem2):
    c = lax.axis_index("c")
    s = lax.axis_index("s")
    pltpu.sync_copy(zz_hbm.at[c, pl.ds(320 * s, 320)], z_sp.at[pl.ds(320 * s, 320)])
    pltpu.sync_copy(cl_hbm.at[pl.ds(5 * s, 5)], clbuf)
    pltpu.sync_copy(dinvb_hbm.at[pl.ds(640 * s, 640)], dbb)
    estart = 156 * s + jnp.minimum(s, 4)
    pltpu.sync_copy(row_hbm.at[pl.ds(estart, 156)], rbuf.at[pl.ds(0, 156)])
    pltpu.sync_copy(col_hbm.at[pl.ds(estart, 156)], cbuf.at[pl.ds(0, 156)])

    @pl.when(s < 4)
    def _():
        pltpu.sync_copy(row_hbm.at[pl.ds(estart + 156, 1)], rbuf.at[pl.ds(156, 1)])
        pltpu.sync_copy(col_hbm.at[pl.ds(estart + 156, 1)], cbuf.at[pl.ds(156, 1)])

    plsc.subcore_barrier()

    for j in range(5):
        pltpu.async_copy(z_sp.at[clbuf.at[j]], gbuf, sem).wait()

        def _mb(r, carry):
            rr = 128 * j + r
            for q in range(4):
                ybuf[r, pl.ds(16 * q, 16)] = (
                    gbuf[r, pl.ds(16 * q, 16)] * dbb[rr, pl.ds(16 * q, 16)])
            return carry

        lax.fori_loop(0, 128, _mb, 0)
        pltpu.sync_copy(ybuf, y_sp.at[pl.ds(640 * s + 128 * j, 128)])
        pltpu.sync_copy(ybuf, acc_sp.at[pl.ds(640 * s + 128 * j, 128)])

    plsc.subcore_barrier()

    nj = 156 + jnp.where(s < 4, 1, 0)

    def _eb(j, carry):
        pltpu.async_copy(y_sp.at[rbuf.at[j]], egbuf, sem2).wait()
        pltpu.sync_copy(egbuf, acc_sp.at[cbuf.at[j]], add=True)
        return carry

    lax.fori_loop(0, nj, _eb, 0)
    plsc.subcore_barrier()
    pltpu.sync_copy(acc_sp.at[pl.ds(640 * s, 640)], acc_hbm.at[c, pl.ds(640 * s, 640)])


_edge_call = pl.kernel(
    _edge_body,
    out_type=jax.ShapeDtypeStruct((NC, NP, CH), jnp.float32),
    mesh=_mesh,
    scratch_types=[
        pltpu.VMEM_SHARED((NPP, CH), jnp.float32),
        pltpu.VMEM_SHARED((NP, CH), jnp.float32),
        pltpu.VMEM_SHARED((NP, CH), jnp.float32),
        pltpu.VMEM((5, 128), jnp.int32),
        pltpu.VMEM((640, CH), jnp.float32),
        pltpu.VMEM((157, 128), jnp.int32),
        pltpu.VMEM((157, 128), jnp.int32),
        pltpu.VMEM((128, CH), jnp.float32),
        pltpu.VMEM((128, CH), jnp.float32),
        pltpu.VMEM((128, CH), jnp.float32),
        pltpu.SemaphoreType.DMA,
        pltpu.SemaphoreType.DMA,
    ],
)


def _final_body(acc_ref, dinv_ref, bc_ref, out_ref):
    dinv = dinv_ref[0:N, :]
    out_ref[:, 0:CH] = dinv * acc_ref[0, 0:N, :] + bc_ref[:, 0:CH]
    out_ref[:, CH:C] = dinv * acc_ref[1, 0:N, :] + bc_ref[:, CH:C]


_final_call = pl.pallas_call(
    _final_body,
    out_shape=jax.ShapeDtypeStruct((N, C), jnp.float32),
)


def kernel(x, edge_index, batch, cluster, unpool_edge_index, unpool_batch,
           new_edge_score, W1, b1, gamma1, beta1, Wc, bc):
    row_r = unpool_edge_index[0].reshape(EROW, 128)
    col_r = unpool_edge_index[1].reshape(EROW, 128)
    cl_p = jnp.concatenate(
        [cluster, jnp.full((NP - N,), PAD_IDX, jnp.int32)]).reshape(NROW, 128)

    fdeg, fcc = _hist_call(col_r, cl_p)
    zz, dinvb, dinv = _dense_call(
        x, new_edge_score.reshape(N_POOLED, 1), W1, b1.reshape(1, C),
        gamma1.reshape(1, C), beta1.reshape(1, C), Wc,
        fcc[:N_POOLED].reshape(N_POOLED, 1),
        fdeg[0].reshape(NP, 1), fdeg[1].reshape(NP, 1))
    acc = _edge_call(zz, cl_p, dinvb, row_r, col_r)
    out = _final_call(acc, dinv, bc.reshape(1, C))
    return (out, unpool_edge_index, unpool_batch)


# speedup vs baseline: 21.8977x; 21.8977x over previous
import jax
import jax.numpy as jnp
from jax import lax
from jax.experimental import pallas as pl
from jax.experimental.pallas import tpu as pltpu
from jax.experimental.pallas import tpu_sc as plsc

N_POOLED, N, E, C = 5000, 10000, 320000, 128
NC, NS = 2, 16
CH = C // NC
NP = 10240
NPP = 5120
NROW = NP // 128
EROW = E // 128
PAD_IDX = N_POOLED

_mesh = plsc.VectorSubcoreMesh(
    core_axis_name="c", subcore_axis_name="s", num_cores=NC, num_subcores=NS)


def _hist_body(col_hbm, clus_hbm, fdeg0_hbm, fdeg1_hbm, fcc_hbm,
               histd_sp, histc_sp, ibuf, clbuf, ones_v, zero_v):
    c = lax.axis_index("c")
    s = lax.axis_index("s")
    for k in range(8):
        ones_v[pl.ds(16 * k, 16)] = jnp.full((16,), 1.0, jnp.float32)

    def _zf(i, carry):
        zero_v[pl.ds(16 * i, 16)] = jnp.zeros((16,), jnp.float32)
        return carry

    lax.fori_loop(0, 64, _zf, 0)

    @pl.when(s < 10)
    def _():
        pltpu.sync_copy(zero_v, histd_sp.at[pl.ds(1024 * s, 1024)])

    @pl.when(jnp.logical_and(c == 1, s < 5))
    def _():
        pltpu.sync_copy(zero_v, histc_sp.at[pl.ds(1024 * s, 1024)])

    plsc.subcore_barrier()

    w = 16 * c + s
    start = 80 * w

    @pl.when(w < 31)
    def _():
        pltpu.sync_copy(col_hbm.at[pl.ds(start, 80)], ibuf)

    @pl.when(w == 31)
    def _():
        pltpu.sync_copy(col_hbm.at[pl.ds(2480, 20)], ibuf.at[pl.ds(0, 20)])

    nj = jnp.where(w < 31, 80, 20)

    def _hb(j, carry):
        pltpu.sync_copy(ones_v, histd_sp.at[ibuf.at[j]], add=True)
        return carry

    lax.fori_loop(0, nj, _hb, 0)

    @pl.when(jnp.logical_and(c == 1, s < 10))
    def _():
        pltpu.sync_copy(clus_hbm.at[pl.ds(8 * s, 8)], clbuf)
        for j in range(8):
            pltpu.sync_copy(ones_v, histc_sp.at[clbuf.at[j]], add=True)

    plsc.subcore_barrier()

    @pl.when(jnp.logical_and(c == 0, s < 10))
    def _():
        pltpu.sync_copy(histd_sp.at[pl.ds(1024 * s, 1024)],
                        fdeg0_hbm.at[pl.ds(1024 * s, 1024)])

    @pl.when(jnp.logical_and(c == 1, s < 10))
    def _():
        pltpu.sync_copy(histd_sp.at[pl.ds(1024 * s, 1024)],
                        fdeg1_hbm.at[pl.ds(1024 * s, 1024)])

    @pl.when(jnp.logical_and(c == 1, s < 5))
    def _():
        pltpu.sync_copy(histc_sp.at[pl.ds(1024 * s, 1024)],
                        fcc_hbm.at[pl.ds(1024 * s, 1024)])


_hist_call = pl.kernel(
    _hist_body,
    out_type=[jax.ShapeDtypeStruct((NP,), jnp.float32),
              jax.ShapeDtypeStruct((NP,), jnp.float32),
              jax.ShapeDtypeStruct((NPP,), jnp.float32)],
    mesh=_mesh,
    scratch_types=[
        pltpu.VMEM_SHARED((NP,), jnp.float32),
        pltpu.VMEM_SHARED((NPP,), jnp.float32),
        pltpu.VMEM((80, 128), jnp.int32),
        pltpu.VMEM((8, 128), jnp.int32),
        pltpu.VMEM((128,), jnp.float32),
        pltpu.VMEM((1024,), jnp.float32),
    ],
)


def _dense_body(x_ref, sc_ref, w1_ref, b1_ref, g1_ref, be_ref, wc_ref, cc_ref,
                f0_ref, f1_ref, z_ref, dinvb_ref, dinv_ref):
    xsn = x_ref[...] / sc_ref[...]
    g = jnp.dot(xsn, w1_ref[...], preferred_element_type=jnp.float32) + b1_ref[...]
    cg = cc_ref[...] * g
    mean = jnp.sum(cg, axis=0, keepdims=True) * (1.0 / N)
    e2 = jnp.sum(cg * g, axis=0, keepdims=True) * (1.0 / N)
    var = e2 - mean * mean
    t = (g - mean) * lax.rsqrt(var + 1e-5) * g1_ref[...] + be_ref[...]
    t = jnp.maximum(t, 0.0)
    z = jnp.dot(t, wc_ref[...], preferred_element_type=jnp.float32)
    z_ref[0:N_POOLED, :] = z
    z_ref[N_POOLED:NPP, :] = jnp.zeros((NPP - N_POOLED, C), jnp.float32)
    dinv = lax.rsqrt(f0_ref[...] + f1_ref[...] + 1.0)
    dinv_ref[...] = dinv
    dinvb_ref[...] = jnp.broadcast_to(dinv, (NP, C))


_dense_call = pl.pallas_call(
    _dense_body,
    out_shape=[jax.ShapeDtypeStruct((NPP, C), jnp.float32),
               jax.ShapeDtypeStruct((NP, C), jnp.float32),
               jax.ShapeDtypeStruct((NP, 1), jnp.float32)],
)


def _ybuild_body(z_hbm, cl_hbm, dinvb_hbm, y_hbm, clbuf, gbuf, dbb, sem):
    c = lax.axis_index("c")
    s = lax.axis_index("s")
    w = 2 * s + c

    @pl.when(w < 10)
    def _():
        pltpu.sync_copy(cl_hbm.at[pl.ds(8 * w, 8)], clbuf)
        for j in range(8):
            pltpu.async_copy(z_hbm.at[clbuf.at[j]], gbuf, sem).wait()
            pltpu.sync_copy(dinvb_hbm.at[pl.ds(1024 * w + 128 * j, 128)], dbb)

            def _mb(r, carry):
                for q in range(8):
                    gbuf[r, pl.ds(16 * q, 16)] = (
                        gbuf[r, pl.ds(16 * q, 16)] * dbb[r, pl.ds(16 * q, 16)])
                return carry

            lax.fori_loop(0, 128, _mb, 0)
            pltpu.sync_copy(gbuf, y_hbm.at[pl.ds(1024 * w + 128 * j, 128)])


_ybuild_call = pl.kernel(
    _ybuild_body,
    out_type=jax.ShapeDtypeStruct((NP, C), jnp.float32),
    mesh=_mesh,
    scratch_types=[
        pltpu.VMEM((8, 128), jnp.int32),
        pltpu.VMEM((128, C), jnp.float32),
        pltpu.VMEM((128, C), jnp.float32),
        pltpu.SemaphoreType.DMA,
    ],
)


def _edge_body(y_hbm, row_hbm, col_hbm, acc_hbm,
               acc_sp, rbuf, cbuf, gbuf, sem):
    c = lax.axis_index("c")
    s = lax.axis_index("s")

    def _zf(r, carry):
        for q in range(8):
            gbuf[r, pl.ds(16 * q, 16)] = jnp.zeros((16,), jnp.float32)
        return carry

    lax.fori_loop(0, 128, _zf, 0)
    for j in range(5):
        pltpu.sync_copy(gbuf, acc_sp.at[pl.ds(640 * s + 128 * j, 128)])
    plsc.subcore_barrier()

    base = 1248 * c + jnp.where(s < 12, 80 * s, 72 * s + 96)
    nchunk = jnp.where(s < 12, 10, 9)

    def _echunk(k, carry):
        cstart = base + 8 * k
        pltpu.sync_copy(row_hbm.at[pl.ds(cstart, 8)], rbuf)
        pltpu.sync_copy(col_hbm.at[pl.ds(cstart, 8)], cbuf)
        for jj in range(8):
            pltpu.async_copy(y_hbm.at[rbuf.at[jj]], gbuf, sem).wait()
            pltpu.sync_copy(gbuf, acc_sp.at[cbuf.at[jj]], add=True)
        return carry

    lax.fori_loop(0, nchunk, _echunk, 0)

    @pl.when(jnp.logical_and(c == 1, s == 15))
    def _():
        pltpu.sync_copy(row_hbm.at[pl.ds(2496, 4)], rbuf.at[pl.ds(0, 4)])
        pltpu.sync_copy(col_hbm.at[pl.ds(2496, 4)], cbuf.at[pl.ds(0, 4)])
        for jj in range(4):
            pltpu.async_copy(y_hbm.at[rbuf.at[jj]], gbuf, sem).wait()
            pltpu.sync_copy(gbuf, acc_sp.at[cbuf.at[jj]], add=True)

    plsc.subcore_barrier()
    pltpu.sync_copy(acc_sp.at[pl.ds(640 * s, 640)], acc_hbm.at[c, pl.ds(640 * s, 640)])


_edge_call = pl.kernel(
    _edge_body,
    out_type=jax.ShapeDtypeStruct((NC, NP, C), jnp.float32),
    mesh=_mesh,
    scratch_types=[
        pltpu.VMEM_SHARED((NP, C), jnp.float32),
        pltpu.VMEM((8, 128), jnp.int32),
        pltpu.VMEM((8, 128), jnp.int32),
        pltpu.VMEM((128, C), jnp.float32),
        pltpu.SemaphoreType.DMA,
    ],
)


def _final_body(acc_ref, y_ref, dinv_ref, bc_ref, out_ref):
    dinv = dinv_ref[0:N, :]
    out_ref[...] = dinv * (acc_ref[0, 0:N, :] + acc_ref[1, 0:N, :]
                           + y_ref[0:N, :]) + bc_ref[...]


_final_call = pl.pallas_call(
    _final_body,
    out_shape=jax.ShapeDtypeStruct((N, C), jnp.float32),
)


def kernel(x, edge_index, batch, cluster, unpool_edge_index, unpool_batch,
           new_edge_score, W1, b1, gamma1, beta1, Wc, bc):
    row_r = unpool_edge_index[0].reshape(EROW, 128)
    col_r = unpool_edge_index[1].reshape(EROW, 128)
    cl_p = jnp.concatenate(
        [cluster, jnp.full((NP - N,), PAD_IDX, jnp.int32)]).reshape(NROW, 128)

    fdeg0, fdeg1, fcc = _hist_call(col_r, cl_p)
    z, dinvb, dinv = _dense_call(
        x, new_edge_score.reshape(N_POOLED, 1), W1, b1.reshape(1, C),
        gamma1.reshape(1, C), beta1.reshape(1, C), Wc,
        fcc[:N_POOLED].reshape(N_POOLED, 1),
        fdeg0.reshape(NP, 1), fdeg1.reshape(NP, 1))
    y = _ybuild_call(z, cl_p, dinvb)
    acc = _edge_call(y, row_r, col_r)
    out = _final_call(acc, y, dinv, bc.reshape(1, C))
    return (out, unpool_edge_index, unpool_batch)
